# Initial kernel scaffold; baseline (speedup 1.0000x reference)
import jax
import jax.numpy as jnp
from jax.experimental import pallas as pl


def kernel(values, positive_mask, negative_mask):
    raise NotImplementedError("write your pallas kernel here")


# speedup vs baseline: 47.1908x; 47.1908x over previous
import jax
import jax.numpy as jnp
from jax.experimental import pallas as pl
from jax.experimental.pallas import tpu as pltpu

_RATIO = 3
_MIN_NEG = 0
_ROWS = 64
_COLS = 20000
_RCHUNK = 8
_NCHUNK = _ROWS // _RCHUNK
_INT_MIN = -2147483648


def _hnm_body(v_ref, pm_ref, nm_ref, out_ref, key_ref, pos_ref, rt_ref,
              cnt_ref, take_ref, cand_ref):
    def p0(i, carry):
        npos, nnz = carry
        r0 = i * _RCHUNK
        v = v_ref[pl.ds(r0, _RCHUNK), :]
        pmi = pm_ref[pl.ds(r0, _RCHUNK), :]
        nmi = nm_ref[pl.ds(r0, _RCHUNK), :]
        flat = v * nmi.astype(jnp.float32)
        b = jax.lax.bitcast_convert_type(flat, jnp.int32)
        key = jnp.where(b >= 0, b, b ^ jnp.int32(0x7FFFFFFF))
        key_ref[pl.ds(r0, _RCHUNK), :] = key
        pos_ref[pl.ds(r0, _RCHUNK), :] = jnp.sum(
            v * pmi.astype(jnp.float32), axis=1, keepdims=True)
        npos = npos + jnp.sum(pmi)
        nnz = nnz + jnp.sum((flat != 0.0).astype(jnp.int32))
        return npos, nnz

    num_pos, nnz = jax.lax.fori_loop(
        0, _NCHUNK, p0, (jnp.int32(0), jnp.int32(0)))
    k_sel = jnp.minimum(
        jnp.maximum(jnp.int32(_RATIO) * num_pos, jnp.int32(_MIN_NEG)), nnz)

    def count_ge(c):
        def cb(i, acc):
            k = key_ref[pl.ds(i * _RCHUNK, _RCHUNK), :]
            return acc + jnp.sum((k >= c).astype(jnp.int32))
        return jax.lax.fori_loop(0, _NCHUNK, cb, jnp.int32(0))

    prefix0 = jnp.where(count_ge(jnp.int32(0)) >= k_sel, jnp.int32(0),
                        jnp.int32(_INT_MIN))

    def sb(t, prefix):
        bit = jnp.int32(30) - t
        cand = prefix | (jnp.int32(1) << bit)
        return jnp.where(count_ge(cand) >= k_sel, cand, prefix)

    t_key = jax.lax.fori_loop(0, 31, sb, prefix0)

    def pc(i, g):
        k = key_ref[pl.ds(i * _RCHUNK, _RCHUNK), :]
        rt_ref[pl.ds(i * _RCHUNK, _RCHUNK), :] = jnp.sum(
            (k == t_key).astype(jnp.int32), axis=1, keepdims=True)
        return g + jnp.sum((k > t_key).astype(jnp.int32))

    above = jax.lax.fori_loop(0, _NCHUNK, pc, jnp.int32(0))
    c_take = k_sel - above

    ii = jax.lax.broadcasted_iota(jnp.int32, (_ROWS, _ROWS), 0)
    jj = jax.lax.broadcasted_iota(jnp.int32, (_ROWS, _ROWS), 1)
    tri = (jj < ii).astype(jnp.float32)
    rtf = rt_ref[...].astype(jnp.float32)
    rtb = jnp.broadcast_to(rtf, (_ROWS, 128))
    excl = jax.lax.dot_general(
        tri, rtb, (((1,), (0,)), ((), ())),
        preferred_element_type=jnp.float32)[:, :1]
    take_f = jnp.clip(c_take.astype(jnp.float32) - excl, 0.0, rtf)
    take_ref[...] = take_f.astype(jnp.int32)

    cand_ref[...] = jnp.zeros((_ROWS, 1), jnp.int32)

    def dbit(t, _):
        bitv = jnp.int32(1) << (jnp.int32(14) - t)

        def dc(i, _):
            r0 = i * _RCHUNK
            k = key_ref[pl.ds(r0, _RCHUNK), :]
            tie = k == t_key
            trial = cand_ref[pl.ds(r0, _RCHUNK), :] | bitv
            colio = jax.lax.broadcasted_iota(jnp.int32, (_RCHUNK, _COLS), 1)
            cnt_ref[pl.ds(r0, _RCHUNK), :] = jnp.sum(
                (tie & (colio < trial)).astype(jnp.int32),
                axis=1, keepdims=True)
            return 0

        jax.lax.fori_loop(0, _NCHUNK, dc, 0)
        cand_ref[...] = jnp.where(cnt_ref[...] <= take_ref[...],
                                  cand_ref[...] | bitv, cand_ref[...])
        return 0

    jax.lax.fori_loop(0, 15, dbit, 0)

    def pe(i, _):
        r0 = i * _RCHUNK
        k = key_ref[pl.ds(r0, _RCHUNK), :]
        v = v_ref[pl.ds(r0, _RCHUNK), :]
        cnd = cand_ref[pl.ds(r0, _RCHUNK), :]
        colio = jax.lax.broadcasted_iota(jnp.int32, (_RCHUNK, _COLS), 1)
        sel = (k > t_key) | ((k == t_key) & (colio < cnd))
        neg = jnp.sum(jnp.where(sel, v, 0.0), axis=1, keepdims=True)
        out_ref[pl.ds(r0, _RCHUNK), :] = pos_ref[pl.ds(r0, _RCHUNK), :] + neg
        return 0

    jax.lax.fori_loop(0, _NCHUNK, pe, 0)


def kernel(values, positive_mask, negative_mask):
    out = pl.pallas_call(
        _hnm_body,
        out_shape=jax.ShapeDtypeStruct((_ROWS, 1), jnp.float32),
        scratch_shapes=[
            pltpu.VMEM((_ROWS, _COLS), jnp.int32),
            pltpu.VMEM((_ROWS, 1), jnp.float32),
            pltpu.VMEM((_ROWS, 1), jnp.int32),
            pltpu.VMEM((_ROWS, 1), jnp.int32),
            pltpu.VMEM((_ROWS, 1), jnp.int32),
            pltpu.VMEM((_ROWS, 1), jnp.int32),
        ],
    )(values, positive_mask, negative_mask)
    return out[:, 0]
